# baseline (speedup 1.0000x reference)
import jax
import jax.numpy as jnp
from jax import lax
from jax.experimental import pallas as pl
from jax.experimental.pallas import tpu as pltpu
from jax.experimental.pallas import tpu_sc as plsc

_B = 4
_S = 2048
_D = 768
_NC = 2
_NS = 16
_NW = _NC * _NS
_SW = _S // _NW
_C = 4
_R = _C * _B
_NCHUNK = _SW // _C
_NBUF = 4
_J = _D // 16


def _body(idx_hbm, table_hbm, pos_hbm, out_hbm, idx_s, in_v, pos_v,
          out_v, sem_g, sem_o):
    wid = lax.axis_index("s") * _NC + lax.axis_index("c")
    s_base = wid * _SW

    pltpu.sync_copy(idx_hbm.at[pl.ds(s_base * _B, _SW * _B)], idx_s)

    gathers = {}
    stores = {}

    def start(g):
        slot = g % _NBUF
        gathers[g] = [
            pltpu.async_copy(
                table_hbm.at[idx_s.at[pl.ds(g * _R, _R)]], in_v.at[slot],
                sem_g),
            pltpu.async_copy(
                pos_hbm.at[pl.ds(s_base + g * _C, _C)], pos_v.at[slot],
                sem_g),
        ]

    for g in range(_NBUF):
        start(g)
    for g in range(_NCHUNK):
        slot = g % _NBUF
        for cp in gathers.pop(g):
            cp.wait()
        if g >= _NBUF:
            for cp in stores.pop(g - _NBUF):
                cp.wait()

        def add_s(s, c, slot=slot):
            @plsc.parallel_loop(0, _J, 1, unroll=8)
            def add_j(j):
                sl = pl.ds(j * 16, 16)
                p = pos_v[slot, s, sl]
                for b in range(_B):
                    out_v[slot, b, s, sl] = in_v[slot, s * _B + b, sl] + p
            return c

        lax.fori_loop(0, _C, add_s, 0)

        stores[g] = [
            pltpu.async_copy(
                out_v.at[slot, b],
                out_hbm.at[b, pl.ds(s_base + g * _C, _C)], sem_o)
            for b in range(_B)
        ]
        if g + _NBUF < _NCHUNK:
            start(g + _NBUF)
    for g in range(_NCHUNK - _NBUF, _NCHUNK):
        for cp in stores.pop(g):
            cp.wait()


@jax.jit
def kernel(inputs, embed_table, pos_table):
    idx = jnp.transpose(inputs).reshape(_S * _B).astype(jnp.int32)
    mesh = plsc.VectorSubcoreMesh(core_axis_name="c", subcore_axis_name="s")
    out = pl.kernel(
        _body,
        out_type=jax.ShapeDtypeStruct((_B, _S, _D), jnp.float32),
        mesh=mesh,
        scratch_types=[
            pltpu.VMEM((_SW * _B,), jnp.int32),
            pltpu.VMEM((_NBUF, _R, _D), jnp.float32),
            pltpu.VMEM((_NBUF, _C, _D), jnp.float32),
            pltpu.VMEM((_NBUF, _B, _C, _D), jnp.float32),
            pltpu.SemaphoreType.DMA,
            pltpu.SemaphoreType.DMA,
        ],
    )(idx, embed_table, pos_table)
    return out


# speedup vs baseline: 1.0667x; 1.0667x over previous
import jax
import jax.numpy as jnp
from jax import lax
from jax.experimental import pallas as pl
from jax.experimental.pallas import tpu as pltpu
from jax.experimental.pallas import tpu_sc as plsc

_B = 4
_S = 2048
_D = 768
_NC = 2
_NS = 16
_NW = _NC * _NS
_SW = _S // _NW
_C = 8
_R = _C * _B
_NCHUNK = _SW // _C
_NBUF = 4
_J = _D // 16


def _body(idx_hbm, table_hbm, pos_hbm, out_hbm, idx_s, oidx, in_v, pos_v,
          sem_g, sem_o):
    wid = lax.axis_index("s") * _NC + lax.axis_index("c")
    s_base = wid * _SW

    pltpu.sync_copy(idx_hbm.at[pl.ds(s_base * _B, _SW * _B)], idx_s)

    lane = jnp.arange(16, dtype=jnp.int32)
    for h in range(2):
        t = lane + 16 * h
        pat = ((t & (_B - 1)) << 11) + (t >> 2) + s_base
        for g in range(_NCHUNK):
            oidx[g, pl.ds(h * 16, 16)] = pat + g * _C

    gathers = {}
    scatters = {}

    def start(g):
        slot = g % _NBUF
        gathers[g] = [
            pltpu.async_copy(
                table_hbm.at[idx_s.at[pl.ds(g * _R, _R)]], in_v.at[slot],
                sem_g),
            pltpu.async_copy(
                pos_hbm.at[pl.ds(s_base + g * _C, _C)], pos_v.at[slot],
                sem_g),
        ]

    start(0)
    start(1)
    start(2)
    for g in range(_NCHUNK):
        slot = g % _NBUF
        for cp in gathers.pop(g):
            cp.wait()

        def add_s(s, c, slot=slot):
            @plsc.parallel_loop(0, _J, 1, unroll=8)
            def add_j(j):
                sl = pl.ds(j * 16, 16)
                p = pos_v[slot, s, sl]
                for b in range(_B):
                    r = s * _B + b
                    in_v[slot, r, sl] = in_v[slot, r, sl] + p
            return c

        lax.fori_loop(0, _C, add_s, 0)

        scatters[g] = pltpu.async_copy(
            in_v.at[slot], out_hbm.at[oidx.at[g]], sem_o)
        if g + _NBUF - 1 < _NCHUNK:
            if g >= 1:
                scatters.pop(g - 1).wait()
            start(g + _NBUF - 1)
    for g in sorted(scatters):
        scatters[g].wait()


@jax.jit
def kernel(inputs, embed_table, pos_table):
    idx = jnp.transpose(inputs).reshape(_S * _B).astype(jnp.int32)
    mesh = plsc.VectorSubcoreMesh(core_axis_name="c", subcore_axis_name="s")
    out = pl.kernel(
        _body,
        out_type=jax.ShapeDtypeStruct((_B * _S, _D), jnp.float32),
        mesh=mesh,
        scratch_types=[
            pltpu.VMEM((_SW * _B,), jnp.int32),
            pltpu.VMEM((_NCHUNK, 2 * 16), jnp.int32),
            pltpu.VMEM((_NBUF, _R, _D), jnp.float32),
            pltpu.VMEM((_NBUF, _C, _D), jnp.float32),
            pltpu.SemaphoreType.DMA,
            pltpu.SemaphoreType.DMA,
        ],
    )(idx, embed_table, pos_table)
    return out.reshape(_B, _S, _D)
